# baseline (speedup 1.0000x reference)
import jax
import jax.numpy as jnp
from jax.experimental import pallas as pl

_BLK = 512


def _body(u_ref, d1_ref, d2_ref, v1_ref, v2_ref, o_ref):
    u = u_ref[:]
    p2 = jnp.minimum(d2_ref[:], jnp.maximum(u, 0.0))
    p1 = jnp.minimum(d1_ref[:], jnp.maximum(u - p2, 0.0))
    o_ref[:] = v2_ref[:] * p2 + v1_ref[:] * p1


def kernel(u, d1, d2, v1, v2):
    B, R = v1.shape
    grid = (B // _BLK,)
    scal_spec = pl.BlockSpec((_BLK, 1), lambda i: (i, 0))
    vec_spec = pl.BlockSpec((_BLK, R), lambda i: (i, 0))
    return pl.pallas_call(
        _body,
        grid=grid,
        in_specs=[scal_spec, scal_spec, scal_spec, vec_spec, vec_spec],
        out_specs=vec_spec,
        out_shape=jax.ShapeDtypeStruct((B, R), v1.dtype),
    )(u, d1, d2, v1, v2)


# speedup vs baseline: 1.3726x; 1.3726x over previous
import jax
import jax.numpy as jnp
from jax.experimental import pallas as pl

_BLK = 4096


def _body(u_ref, d1_ref, d2_ref, v1_ref, v2_ref, o_ref):
    u = u_ref[:]
    p2 = jnp.minimum(d2_ref[:], jnp.maximum(u, 0.0))
    p1 = jnp.minimum(d1_ref[:], jnp.maximum(u - p2, 0.0))
    o_ref[:] = v2_ref[:] * p2 + v1_ref[:] * p1


def kernel(u, d1, d2, v1, v2):
    B, R = v1.shape
    grid = (B // _BLK,)
    scal_spec = pl.BlockSpec((_BLK, 1), lambda i: (i, 0))
    vec_spec = pl.BlockSpec((_BLK, R), lambda i: (i, 0))
    return pl.pallas_call(
        _body,
        grid=grid,
        in_specs=[scal_spec, scal_spec, scal_spec, vec_spec, vec_spec],
        out_specs=vec_spec,
        out_shape=jax.ShapeDtypeStruct((B, R), v1.dtype),
    )(u, d1, d2, v1, v2)
